# baseline (speedup 1.0000x reference)
import jax
import jax.numpy as jnp
from jax.experimental import pallas as pl
from jax.experimental.pallas import tpu as pltpu

BLOCK_ROWS = 25000


def _fused_linear_elu(x_ref, w_ref, b_ref, o_ref):
    y = jnp.dot(x_ref[...], w_ref[...], preferred_element_type=jnp.float32)
    y = y + b_ref[...]
    o_ref[...] = jnp.where(y > 0, y, jnp.exp(jnp.minimum(y, 0.0)) - 1.0)


def kernel(features_list, metapath_instances_list, edge_types, W0, b0):
    instances = metapath_instances_list[0]
    n, d_in = instances.shape
    d_out = W0.shape[0]
    wt = W0.T
    b = b0.reshape(1, d_out)

    grid = (n // BLOCK_ROWS,)
    return pl.pallas_call(
        _fused_linear_elu,
        grid=grid,
        in_specs=[
            pl.BlockSpec((BLOCK_ROWS, d_in), lambda i: (i, 0)),
            pl.BlockSpec((d_in, d_out), lambda i: (0, 0)),
            pl.BlockSpec((1, d_out), lambda i: (0, 0)),
        ],
        out_specs=pl.BlockSpec((BLOCK_ROWS, d_out), lambda i: (i, 0)),
        out_shape=jax.ShapeDtypeStruct((n, d_out), jnp.float32),
        compiler_params=pltpu.CompilerParams(
            dimension_semantics=("parallel",),
        ),
    )(instances, wt, b)


# speedup vs baseline: 1.0276x; 1.0276x over previous
import jax
import jax.numpy as jnp
from jax.experimental import pallas as pl
from jax.experimental.pallas import tpu as pltpu

BLOCK_ROWS = 20000


def _fused_linear_elu(x_ref, w_ref, b_ref, o_ref):
    y = jnp.dot(x_ref[...], w_ref[...], preferred_element_type=jnp.float32)
    y = y + b_ref[...]
    o_ref[...] = jnp.where(y > 0, y, jnp.exp(y) - 1.0)


def kernel(features_list, metapath_instances_list, edge_types, W0, b0):
    instances = metapath_instances_list[0]
    n, d_in = instances.shape
    d_out = W0.shape[0]
    wt = W0.T
    b = b0.reshape(1, d_out)

    grid = (n // BLOCK_ROWS,)
    return pl.pallas_call(
        _fused_linear_elu,
        grid=grid,
        in_specs=[
            pl.BlockSpec((BLOCK_ROWS, d_in), lambda i: (i, 0)),
            pl.BlockSpec((d_in, d_out), lambda i: (0, 0)),
            pl.BlockSpec((1, d_out), lambda i: (0, 0)),
        ],
        out_specs=pl.BlockSpec((BLOCK_ROWS, d_out), lambda i: (i, 0)),
        out_shape=jax.ShapeDtypeStruct((n, d_out), jnp.float32),
        compiler_params=pltpu.CompilerParams(
            dimension_semantics=("parallel",),
        ),
    )(instances, wt, b)
